# initial kernel scaffold (unmeasured)
import jax
import jax.numpy as jnp
from jax import lax
from jax.experimental import pallas as pl
from jax.experimental.pallas import tpu as pltpu

N_DEV = 8
B_LOC = 2
SQ = 512
SKV = 512
H_LOC = 8
DH = 64
DM = 768


def kernel(x, Wq, K_ext, V_ext, Wo):
    my = lax.axis_index("i")
    wq_r = Wq.reshape(DM, H_LOC, DH).transpose(1, 0, 2)
    wo_r = Wo.reshape(H_LOC, DH, DM)
    k_loc = lax.dynamic_slice_in_dim(K_ext, B_LOC * my, B_LOC, 0)
    v_loc = lax.dynamic_slice_in_dim(V_ext, B_LOC * my, B_LOC, 0)
    k_t = jnp.transpose(k_loc, (2, 0, 1, 3))
    v_t = jnp.transpose(v_loc, (2, 0, 1, 3))

    def body(x_ref, wq_ref, wo_ref, k_ref, v_ref, out_ref,
             wq_comm, wo_comm, wq_send, wq_recv, wo_send, wo_recv):
        my_i = lax.axis_index("i")
        left = lax.rem(my_i + N_DEV - 1, N_DEV)
        right = lax.rem(my_i + 1, N_DEV)

        barrier = pltpu.get_barrier_semaphore()
        pl.semaphore_signal(barrier, inc=1, device_id=(left,),
                            device_id_type=pl.DeviceIdType.MESH)
        pl.semaphore_wait(barrier, 1)

        wq_comm[0] = wq_ref[...]
        wo_comm[0] = wo_ref[...]

        qb = lax.broadcasted_iota(jnp.int32, (SQ, SKV), 0) // 64
        kb = lax.broadcasted_iota(jnp.int32, (SQ, SKV), 1) // 64
        mask = (qb == kb) | (kb == 0) | (lax.rem(qb + kb, 3) == 0)

        for h in range(N_DEV):
            slot = h % 2
            nxt = (h + 1) % 2
            rdmas = []
            if h < N_DEV - 1:
                for buf, ssem, rsem in ((wq_comm, wq_send, wq_recv),
                                        (wo_comm, wo_send, wo_recv)):
                    r = pltpu.make_async_remote_copy(
                        src_ref=buf.at[slot], dst_ref=buf.at[nxt],
                        send_sem=ssem.at[slot], recv_sem=rsem.at[nxt],
                        device_id=(right,),
                        device_id_type=pl.DeviceIdType.MESH)
                    r.start()
                    rdmas.append(r)

            origin = lax.rem(my_i - h + N_DEV, N_DEV)
            for b in range(B_LOC):
                xb = x_ref[b]

                def head_body(hh, acc):
                    g = origin * H_LOC + hh
                    q = jnp.dot(xb, wq_comm[slot, hh],
                                preferred_element_type=jnp.float32)
                    k = k_ref[g, b]
                    s = lax.dot_general(
                        q, k, (((1,), (1,)), ((), ())),
                        preferred_element_type=jnp.float32) * 0.125
                    s = jnp.where(mask, s, -1e9)
                    m = jnp.max(s, axis=1, keepdims=True)
                    e = jnp.exp(s - m)
                    w = e / jnp.sum(e, axis=1, keepdims=True)
                    ctx = jnp.dot(w, v_ref[g, b],
                                  preferred_element_type=jnp.float32)
                    return acc + jnp.dot(ctx, wo_comm[slot, hh],
                                         preferred_element_type=jnp.float32)

                acc = lax.fori_loop(0, H_LOC, head_body,
                                    jnp.zeros((SQ, DM), jnp.float32))
                if h == 0:
                    out_ref[b] = acc
                else:
                    out_ref[b] = out_ref[b] + acc

            for r in rdmas:
                r.wait()

    return pl.pallas_call(
        body,
        out_shape=jax.ShapeDtypeStruct((B_LOC, SQ, DM), jnp.float32),
        in_specs=[pl.BlockSpec(memory_space=pltpu.VMEM)] * 5,
        out_specs=pl.BlockSpec(memory_space=pltpu.VMEM),
        scratch_shapes=[
            pltpu.VMEM((2, H_LOC, DM, DH), jnp.float32),
            pltpu.VMEM((2, H_LOC, DH, DM), jnp.float32),
            pltpu.SemaphoreType.DMA((2,)),
            pltpu.SemaphoreType.DMA((2,)),
            pltpu.SemaphoreType.DMA((2,)),
            pltpu.SemaphoreType.DMA((2,)),
        ],
        compiler_params=pltpu.CompilerParams(collective_id=0),
    )(x, wq_r, wo_r, k_t, v_t)


# baseline (device time: 330430 ns/iter reference)
import jax
import jax.numpy as jnp
from jax import lax
from jax.experimental import pallas as pl
from jax.experimental.pallas import tpu as pltpu

N_DEV = 8
B_LOC = 2
SQ = 512
SKV = 512
H_LOC = 8
DH = 64
DM = 768


def kernel(x, Wq, K_ext, V_ext, Wo):
    my = lax.axis_index("i")
    wq_r = Wq.reshape(DM, H_LOC, DH).transpose(1, 2, 0)
    wo_r = Wo.reshape(H_LOC, DH, DM)
    k_loc = lax.dynamic_slice_in_dim(K_ext, B_LOC * my, B_LOC, 0)
    v_loc = lax.dynamic_slice_in_dim(V_ext, B_LOC * my, B_LOC, 0)
    k_t = jnp.transpose(k_loc, (2, 0, 3, 1))
    v_t = jnp.transpose(v_loc, (2, 0, 3, 1))

    def body(x_ref, wq_ref, wo_ref, k_ref, v_ref, out_ref,
             wq_comm, wo_comm, wq_send, wq_recv, wo_send, wo_recv):
        my_i = lax.axis_index("i")
        left = lax.rem(my_i + N_DEV - 1, N_DEV)
        right = lax.rem(my_i + 1, N_DEV)

        barrier = pltpu.get_barrier_semaphore()
        pl.semaphore_signal(barrier, inc=1, device_id=(left,),
                            device_id_type=pl.DeviceIdType.MESH)
        pl.semaphore_wait(barrier, 1)

        wq_comm[0] = wq_ref[...]
        wo_comm[0] = wo_ref[...]

        qb = lax.broadcasted_iota(jnp.int32, (SQ, SKV), 0) // 64
        kb = lax.broadcasted_iota(jnp.int32, (SQ, SKV), 1) // 64
        mask = (qb == kb) | (kb == 0) | (lax.rem(qb + kb, 3) == 0)

        for h in range(N_DEV):
            slot = h % 2
            nxt = (h + 1) % 2
            rdmas = []
            if h < N_DEV - 1:
                for buf, ssem, rsem in ((wq_comm, wq_send, wq_recv),
                                        (wo_comm, wo_send, wo_recv)):
                    r = pltpu.make_async_remote_copy(
                        src_ref=buf.at[slot], dst_ref=buf.at[nxt],
                        send_sem=ssem.at[slot], recv_sem=rsem.at[nxt],
                        device_id=(right,),
                        device_id_type=pl.DeviceIdType.MESH)
                    r.start()
                    rdmas.append(r)

            origin = lax.rem(my_i - h + N_DEV, N_DEV)
            for b in range(B_LOC):
                xb = x_ref[b]

                def head_body(hh, acc):
                    g = origin * H_LOC + hh
                    qt = lax.dot_general(
                        wq_comm[slot, hh], xb, (((1,), (1,)), ((), ())),
                        preferred_element_type=jnp.float32)
                    s = lax.dot_general(
                        qt, k_ref[g, b], (((0,), (0,)), ((), ())),
                        preferred_element_type=jnp.float32) * 0.125
                    s = jnp.where(mask, s, -1e9)
                    m = jnp.max(s, axis=1, keepdims=True)
                    e = jnp.exp(s - m)
                    w = e / jnp.sum(e, axis=1, keepdims=True)
                    ctxt = lax.dot_general(
                        v_ref[g, b], w, (((1,), (1,)), ((), ())),
                        preferred_element_type=jnp.float32)
                    return acc + lax.dot_general(
                        ctxt, wo_comm[slot, hh], (((0,), (0,)), ((), ())),
                        preferred_element_type=jnp.float32)

                acc = lax.fori_loop(0, H_LOC, head_body,
                                    jnp.zeros((SQ, DM), jnp.float32))
                if h == 0:
                    out_ref[b] = acc
                else:
                    out_ref[b] = out_ref[b] + acc

            for r in rdmas:
                r.wait()

    return pl.pallas_call(
        body,
        out_shape=jax.ShapeDtypeStruct((B_LOC, SQ, DM), jnp.float32),
        in_specs=[pl.BlockSpec(memory_space=pltpu.VMEM)] * 5,
        out_specs=pl.BlockSpec(memory_space=pltpu.VMEM),
        scratch_shapes=[
            pltpu.VMEM((2, H_LOC, DH, DM), jnp.float32),
            pltpu.VMEM((2, H_LOC, DH, DM), jnp.float32),
            pltpu.SemaphoreType.DMA((2,)),
            pltpu.SemaphoreType.DMA((2,)),
            pltpu.SemaphoreType.DMA((2,)),
            pltpu.SemaphoreType.DMA((2,)),
        ],
        compiler_params=pltpu.CompilerParams(
            collective_id=0,
            vmem_limit_bytes=60 * 1024 * 1024,
        ),
    )(x, wq_r, wo_r, k_t, v_t)


# device time: 218452 ns/iter; 1.5126x vs baseline; 1.5126x over previous
import jax
import jax.numpy as jnp
from jax import lax
from jax.experimental import pallas as pl
from jax.experimental.pallas import tpu as pltpu

N_DEV = 8
B_LOC = 2
SQ = 512
SKV = 512
H_LOC = 8
DH = 64
DM = 768

_MESH = pl.DeviceIdType.MESH


def kernel(x, Wq, K_ext, V_ext, Wo):
    my = lax.axis_index("i")
    x_s = x * jnp.float32(0.125)
    wq_t = Wq.reshape(DM, H_LOC * DH).T
    wqo = jnp.stack([wq_t, Wo])
    k_loc = lax.dynamic_slice_in_dim(K_ext, B_LOC * my, B_LOC, 0)
    v_loc = lax.dynamic_slice_in_dim(V_ext, B_LOC * my, B_LOC, 0)
    k_t = jnp.transpose(k_loc, (2, 0, 3, 1))
    v_t = jnp.transpose(v_loc, (2, 0, 3, 1))

    def body(x_ref, wqo_ref, k_ref, v_ref, out_ref,
             rbuf, lbuf, qs, cx, rsend, rrecv, lsend, lrecv, rcred, lcred):
        my_i = lax.axis_index("i")
        left = lax.rem(my_i + N_DEV - 1, N_DEV)
        right = lax.rem(my_i + 1, N_DEV)

        barrier = pltpu.get_barrier_semaphore()
        for nbr in (left, right):
            pl.semaphore_signal(barrier, inc=1, device_id=(nbr,),
                                device_id_type=_MESH)
        pl.semaphore_wait(barrier, 2)

        rbuf[0] = wqo_ref[...]
        lbuf[0] = wqo_ref[...]

        qb = lax.broadcasted_iota(jnp.int32, (SQ, SKV), 0) // 64
        kb = lax.broadcasted_iota(jnp.int32, (SQ, SKV), 1) // 64
        keep = (qb == kb) | (kb == 0) | (lax.rem(qb + kb, 3) == 0)
        bias = jnp.where(keep, jnp.float32(0.0), jnp.float32(-1e9))

        def compute_group(grp, buf, slot, first):
            for b in range(B_LOC):
                xb = x_ref[b]
                qs[...] = lax.dot_general(
                    buf[slot, 0], xb, (((1,), (1,)), ((), ())),
                    preferred_element_type=jnp.float32)

                def head(hh, carry):
                    g = grp * H_LOC + hh
                    qt = qs[pl.ds(hh * DH, DH), :]
                    s = lax.dot_general(
                        qt, k_ref[g, b], (((0,), (0,)), ((), ())),
                        preferred_element_type=jnp.float32)
                    s = s + bias
                    m = jnp.max(s, axis=1, keepdims=True)
                    e = jnp.exp(s - m)
                    w = e / jnp.sum(e, axis=1, keepdims=True)
                    cx[pl.ds(hh * DH, DH), :] = lax.dot_general(
                        v_ref[g, b], w, (((1,), (1,)), ((), ())),
                        preferred_element_type=jnp.float32)
                    return carry

                lax.fori_loop(0, H_LOC, head, 0)
                contrib = lax.dot_general(
                    cx[...], buf[slot, 1], (((0,), (0,)), ((), ())),
                    preferred_element_type=jnp.float32)
                if first:
                    out_ref[b] = contrib
                else:
                    out_ref[b] = out_ref[b] + contrib

        for r in range(5):
            rs = r % 2
            ns = (r + 1) % 2
            started = []
            if r < 4:
                if r >= 1:
                    pl.semaphore_wait(rcred, 1)
                rr = pltpu.make_async_remote_copy(
                    src_ref=rbuf.at[rs], dst_ref=rbuf.at[ns],
                    send_sem=rsend.at[rs], recv_sem=rrecv.at[ns],
                    device_id=(right,), device_id_type=_MESH)
                rr.start()
                started.append(rr)
            if r < 3:
                if r >= 1:
                    pl.semaphore_wait(lcred, 1)
                lr = pltpu.make_async_remote_copy(
                    src_ref=lbuf.at[rs], dst_ref=lbuf.at[ns],
                    send_sem=lsend.at[rs], recv_sem=lrecv.at[ns],
                    device_id=(left,), device_id_type=_MESH)
                lr.start()
                started.append(lr)

            compute_group(lax.rem(my_i - r + N_DEV, N_DEV), rbuf, rs,
                          first=(r == 0))
            if 1 <= r <= 3:
                compute_group(lax.rem(my_i + r, N_DEV), lbuf, rs, first=False)

            for rd in started:
                rd.wait()
            if r <= 2:
                pl.semaphore_signal(rcred, inc=1, device_id=(left,),
                                    device_id_type=_MESH)
            if r <= 1:
                pl.semaphore_signal(lcred, inc=1, device_id=(right,),
                                    device_id_type=_MESH)

    return pl.pallas_call(
        body,
        out_shape=jax.ShapeDtypeStruct((B_LOC, SQ, DM), jnp.float32),
        in_specs=[pl.BlockSpec(memory_space=pltpu.VMEM)] * 4,
        out_specs=pl.BlockSpec(memory_space=pltpu.VMEM),
        scratch_shapes=[
            pltpu.VMEM((2, 2, H_LOC * DH, DM), jnp.float32),
            pltpu.VMEM((2, 2, H_LOC * DH, DM), jnp.float32),
            pltpu.VMEM((H_LOC * DH, SQ), jnp.float32),
            pltpu.VMEM((H_LOC * DH, SQ), jnp.float32),
            pltpu.SemaphoreType.DMA((2,)),
            pltpu.SemaphoreType.DMA((2,)),
            pltpu.SemaphoreType.DMA((2,)),
            pltpu.SemaphoreType.DMA((2,)),
            pltpu.SemaphoreType.REGULAR,
            pltpu.SemaphoreType.REGULAR,
        ],
        compiler_params=pltpu.CompilerParams(
            collective_id=0,
            vmem_limit_bytes=62 * 1024 * 1024,
        ),
    )(x_s, wqo, k_t, v_t)


# device time: 172275 ns/iter; 1.9180x vs baseline; 1.2680x over previous
import jax
import jax.numpy as jnp
from jax import lax
from jax.experimental import pallas as pl
from jax.experimental.pallas import tpu as pltpu

N_DEV = 8
B_LOC = 2
SQ = 512
SKV = 512
H_LOC = 8
DH = 64
DM = 768

_MESH = pl.DeviceIdType.MESH


def kernel(x, Wq, K_ext, V_ext, Wo):
    my = lax.axis_index("i")
    x_s = x * jnp.float32(0.125)
    wq_t = Wq.reshape(DM, H_LOC * DH).T
    wqo = jnp.stack([wq_t, Wo])
    k_loc = lax.dynamic_slice_in_dim(K_ext, B_LOC * my, B_LOC, 0)
    v_loc = lax.dynamic_slice_in_dim(V_ext, B_LOC * my, B_LOC, 0)
    k_t = jnp.transpose(k_loc, (2, 0, 3, 1))
    v_t = jnp.transpose(v_loc, (2, 0, 3, 1))

    def body(x_ref, wqo_ref, k_ref, v_ref, out_ref,
             rbuf, lbuf, qs, cx, rsend, rrecv, lsend, lrecv, rcred, lcred):
        my_i = lax.axis_index("i")
        left = lax.rem(my_i + N_DEV - 1, N_DEV)
        right = lax.rem(my_i + 1, N_DEV)

        rbuf[0] = wqo_ref[...]
        lbuf[0] = wqo_ref[...]

        qb = lax.broadcasted_iota(jnp.int32, (SQ, SKV), 0) // 64
        kb = lax.broadcasted_iota(jnp.int32, (SQ, SKV), 1) // 64
        keep = (qb == kb) | (kb == 0) | (lax.rem(qb + kb, 3) == 0)
        bias = jnp.where(keep, jnp.float32(0.0), jnp.float32(-1e9))

        def compute_group(grp, buf, slot, first):
            for b in range(B_LOC):
                xb = x_ref[b]
                qs[...] = lax.dot_general(
                    buf[slot, 0], xb, (((1,), (1,)), ((), ())),
                    preferred_element_type=jnp.float32)

                def head(hh, carry):
                    g = grp * H_LOC + hh
                    qt = qs[pl.ds(hh * DH, DH), :]
                    s = lax.dot_general(
                        qt, k_ref[g, b], (((0,), (0,)), ((), ())),
                        preferred_element_type=jnp.float32)
                    s = s + bias
                    m = jnp.max(s, axis=1, keepdims=True)
                    e = jnp.exp(s - m)
                    w = e / jnp.sum(e, axis=1, keepdims=True)
                    cx[pl.ds(hh * DH, DH), :] = lax.dot_general(
                        v_ref[g, b], w, (((1,), (1,)), ((), ())),
                        preferred_element_type=jnp.float32)
                    return carry

                lax.fori_loop(0, H_LOC, head, 0)
                contrib = lax.dot_general(
                    cx[...], buf[slot, 1], (((0,), (0,)), ((), ())),
                    preferred_element_type=jnp.float32)
                if first:
                    out_ref[b] = contrib
                else:
                    out_ref[b] = out_ref[b] + contrib

        for r in range(8):
            compute_group(lax.rem(my_i - r + N_DEV, N_DEV), rbuf, 0,
                          first=(r == 0))

    return pl.pallas_call(
        body,
        out_shape=jax.ShapeDtypeStruct((B_LOC, SQ, DM), jnp.float32),
        in_specs=[pl.BlockSpec(memory_space=pltpu.VMEM)] * 4,
        out_specs=pl.BlockSpec(memory_space=pltpu.VMEM),
        scratch_shapes=[
            pltpu.VMEM((2, 2, H_LOC * DH, DM), jnp.float32),
            pltpu.VMEM((2, 2, H_LOC * DH, DM), jnp.float32),
            pltpu.VMEM((H_LOC * DH, SQ), jnp.float32),
            pltpu.VMEM((H_LOC * DH, SQ), jnp.float32),
            pltpu.SemaphoreType.DMA((2,)),
            pltpu.SemaphoreType.DMA((2,)),
            pltpu.SemaphoreType.DMA((2,)),
            pltpu.SemaphoreType.DMA((2,)),
            pltpu.SemaphoreType.REGULAR,
            pltpu.SemaphoreType.REGULAR,
        ],
        compiler_params=pltpu.CompilerParams(
            vmem_limit_bytes=62 * 1024 * 1024,
        ),
    )(x_s, wqo, k_t, v_t)


# device time: 147977 ns/iter; 2.2330x vs baseline; 1.1642x over previous
import jax
import jax.numpy as jnp
from jax import lax
from jax.experimental import pallas as pl
from jax.experimental.pallas import tpu as pltpu

N_DEV = 8
B_LOC = 2
SQ = 512
SKV = 512
H_LOC = 8
DH = 64
DM = 768

_MESH = pl.DeviceIdType.MESH


def kernel(x, Wq, K_ext, V_ext, Wo):
    my = lax.axis_index("i")
    x_s = (x * jnp.float32(0.125)).astype(jnp.bfloat16)
    wq_t = Wq.reshape(DM, H_LOC * DH).T
    wqo = jnp.stack([wq_t, Wo]).astype(jnp.bfloat16)
    k_loc = lax.dynamic_slice_in_dim(K_ext, B_LOC * my, B_LOC, 0)
    v_loc = lax.dynamic_slice_in_dim(V_ext, B_LOC * my, B_LOC, 0)
    k_t = jnp.transpose(k_loc, (2, 0, 3, 1)).astype(jnp.bfloat16)
    v_t = jnp.transpose(v_loc, (2, 0, 3, 1)).astype(jnp.bfloat16)

    def body(x_ref, wqo_ref, k_ref, v_ref, out_ref,
             rbuf, lbuf, qs, cx, rsend, rrecv, lsend, lrecv, rcred, lcred):
        my_i = lax.axis_index("i")
        left = lax.rem(my_i + N_DEV - 1, N_DEV)
        right = lax.rem(my_i + 1, N_DEV)

        barrier = pltpu.get_barrier_semaphore()
        for nbr in (left, right):
            pl.semaphore_signal(barrier, inc=1, device_id=(nbr,),
                                device_id_type=_MESH)
        pl.semaphore_wait(barrier, 2)

        rbuf[0] = wqo_ref[...]
        lbuf[0] = wqo_ref[...]

        qb = lax.broadcasted_iota(jnp.int32, (SQ, SKV), 0) // 64
        kb = lax.broadcasted_iota(jnp.int32, (SQ, SKV), 1) // 64
        keep = (qb == kb) | (kb == 0) | (lax.rem(qb + kb, 3) == 0)
        bias = jnp.where(keep, jnp.float32(0.0), jnp.float32(-1e9))

        def compute_group(grp, buf, slot, first):
            for b in range(B_LOC):
                xb = x_ref[b]
                qs[...] = lax.dot_general(
                    buf[slot, 0], xb, (((1,), (1,)), ((), ())),
                    preferred_element_type=jnp.float32,
                ).astype(jnp.bfloat16)

                def head(hh, carry):
                    g = grp * H_LOC + hh
                    qt = qs[pl.ds(hh * DH, DH), :]
                    s = lax.dot_general(
                        qt, k_ref[g, b], (((0,), (0,)), ((), ())),
                        preferred_element_type=jnp.float32)
                    e = jnp.exp(s + bias)
                    denom = jnp.sum(e, axis=1)
                    ctxt = lax.dot_general(
                        v_ref[g, b], e.astype(jnp.bfloat16),
                        (((1,), (1,)), ((), ())),
                        preferred_element_type=jnp.float32)
                    cx[pl.ds(hh * DH, DH), :] = (
                        ctxt / denom[None, :]).astype(jnp.bfloat16)
                    return carry

                lax.fori_loop(0, H_LOC, head, 0)
                contrib = lax.dot_general(
                    cx[...], buf[slot, 1], (((0,), (0,)), ((), ())),
                    preferred_element_type=jnp.float32)
                if first:
                    out_ref[b] = contrib
                else:
                    out_ref[b] = out_ref[b] + contrib

        for r in range(5):
            rs = r % 2
            ns = (r + 1) % 2
            started = []
            if r < 4:
                if r >= 1:
                    pl.semaphore_wait(rcred, 1)
                rr = pltpu.make_async_remote_copy(
                    src_ref=rbuf.at[rs], dst_ref=rbuf.at[ns],
                    send_sem=rsend.at[rs], recv_sem=rrecv.at[ns],
                    device_id=(right,), device_id_type=_MESH)
                rr.start()
                started.append(rr)
            if r < 3:
                if r >= 1:
                    pl.semaphore_wait(lcred, 1)
                lr = pltpu.make_async_remote_copy(
                    src_ref=lbuf.at[rs], dst_ref=lbuf.at[ns],
                    send_sem=lsend.at[rs], recv_sem=lrecv.at[ns],
                    device_id=(left,), device_id_type=_MESH)
                lr.start()
                started.append(lr)

            compute_group(lax.rem(my_i - r + N_DEV, N_DEV), rbuf, rs,
                          first=(r == 0))
            if 1 <= r <= 3:
                compute_group(lax.rem(my_i + r, N_DEV), lbuf, rs, first=False)

            for rd in started:
                rd.wait()
            if r <= 2:
                pl.semaphore_signal(rcred, inc=1, device_id=(left,),
                                    device_id_type=_MESH)
            if r <= 1:
                pl.semaphore_signal(lcred, inc=1, device_id=(right,),
                                    device_id_type=_MESH)

    return pl.pallas_call(
        body,
        out_shape=jax.ShapeDtypeStruct((B_LOC, SQ, DM), jnp.float32),
        in_specs=[pl.BlockSpec(memory_space=pltpu.VMEM)] * 4,
        out_specs=pl.BlockSpec(memory_space=pltpu.VMEM),
        scratch_shapes=[
            pltpu.VMEM((2, 2, H_LOC * DH, DM), jnp.bfloat16),
            pltpu.VMEM((2, 2, H_LOC * DH, DM), jnp.bfloat16),
            pltpu.VMEM((H_LOC * DH, SQ), jnp.bfloat16),
            pltpu.VMEM((H_LOC * DH, SQ), jnp.bfloat16),
            pltpu.SemaphoreType.DMA((2,)),
            pltpu.SemaphoreType.DMA((2,)),
            pltpu.SemaphoreType.DMA((2,)),
            pltpu.SemaphoreType.DMA((2,)),
            pltpu.SemaphoreType.REGULAR,
            pltpu.SemaphoreType.REGULAR,
        ],
        compiler_params=pltpu.CompilerParams(
            collective_id=0,
            vmem_limit_bytes=62 * 1024 * 1024,
        ),
    )(x_s, wqo, k_t, v_t)


# device time: 134084 ns/iter; 2.4644x vs baseline; 1.1036x over previous
import jax
import jax.numpy as jnp
from jax import lax
from jax.experimental import pallas as pl
from jax.experimental.pallas import tpu as pltpu

N_DEV = 8
B_LOC = 2
SQ = 512
SKV = 512
H_LOC = 8
DH = 64
DM = 768

_MESH = pl.DeviceIdType.MESH


def kernel(x, Wq, K_ext, V_ext, Wo):
    my = lax.axis_index("i")
    x_s = (x * jnp.float32(0.125)).astype(jnp.bfloat16)
    wq_t = Wq.reshape(DM, H_LOC * DH).T
    wqo = jnp.stack([wq_t, Wo]).astype(jnp.bfloat16)
    k_loc = lax.dynamic_slice_in_dim(K_ext, B_LOC * my, B_LOC, 0)
    v_loc = lax.dynamic_slice_in_dim(V_ext, B_LOC * my, B_LOC, 0)
    k_t = jnp.transpose(k_loc, (2, 0, 3, 1)).astype(jnp.bfloat16)
    v_t = jnp.transpose(v_loc, (2, 0, 3, 1)).astype(jnp.bfloat16)
    v_t = jnp.concatenate(
        [v_t, jnp.ones((64, B_LOC, 1, SKV), jnp.bfloat16)], axis=2)

    def body(x_ref, wqo_ref, k_ref, v_ref, out_ref,
             rbuf, lbuf, qs, cx, rsend, rrecv, lsend, lrecv, rcred, lcred):
        my_i = lax.axis_index("i")
        left = lax.rem(my_i + N_DEV - 1, N_DEV)
        right = lax.rem(my_i + 1, N_DEV)

        barrier = pltpu.get_barrier_semaphore()
        for nbr in (left, right):
            pl.semaphore_signal(barrier, inc=1, device_id=(nbr,),
                                device_id_type=_MESH)
        pl.semaphore_wait(barrier, 2)

        rbuf[0] = wqo_ref[...]
        lbuf[0] = wqo_ref[...]

        qb = lax.broadcasted_iota(jnp.int32, (SQ, SKV), 0) // 64
        kb = lax.broadcasted_iota(jnp.int32, (SQ, SKV), 1) // 64
        keep = (qb == kb) | (kb == 0) | (lax.rem(qb + kb, 3) == 0)
        bias = jnp.where(keep, jnp.float32(0.0), jnp.float32(-1e9))

        def compute_group(grp, buf, slot, first):
            x_all = x_ref[...].reshape(B_LOC * SQ, DM)
            qs[...] = lax.dot_general(
                buf[slot, 0], x_all, (((1,), (1,)), ((), ())),
                preferred_element_type=jnp.float32,
            ).astype(jnp.bfloat16)

            def head(hh, carry):
                g = grp * H_LOC + hh
                for b in range(B_LOC):
                    qt = qs[pl.ds(hh * DH, DH), b * SQ:(b + 1) * SQ]
                    s = lax.dot_general(
                        qt, k_ref[g, b], (((0,), (0,)), ((), ())),
                        preferred_element_type=jnp.float32)
                    e = jnp.exp(s + bias)
                    ca = lax.dot_general(
                        v_ref[g, b], e.astype(jnp.bfloat16),
                        (((1,), (1,)), ((), ())),
                        preferred_element_type=jnp.float32)
                    cx[pl.ds(hh * DH, DH), b * SQ:(b + 1) * SQ] = (
                        ca[:DH] / ca[DH:DH + 1]).astype(jnp.bfloat16)
                return carry

            lax.fori_loop(0, H_LOC, head, 0)
            contrib = lax.dot_general(
                cx[...], buf[slot, 1], (((0,), (0,)), ((), ())),
                preferred_element_type=jnp.float32)
            for b in range(B_LOC):
                blk = contrib[b * SQ:(b + 1) * SQ]
                if first:
                    out_ref[b] = blk
                else:
                    out_ref[b] = out_ref[b] + blk

        for r in range(5):
            rs = r % 2
            ns = (r + 1) % 2
            started = []
            if r < 4:
                if r >= 1:
                    pl.semaphore_wait(rcred, 1)
                rr = pltpu.make_async_remote_copy(
                    src_ref=rbuf.at[rs], dst_ref=rbuf.at[ns],
                    send_sem=rsend.at[rs], recv_sem=rrecv.at[ns],
                    device_id=(right,), device_id_type=_MESH)
                rr.start()
                started.append(rr)
            if r < 3:
                if r >= 1:
                    pl.semaphore_wait(lcred, 1)
                lr = pltpu.make_async_remote_copy(
                    src_ref=lbuf.at[rs], dst_ref=lbuf.at[ns],
                    send_sem=lsend.at[rs], recv_sem=lrecv.at[ns],
                    device_id=(left,), device_id_type=_MESH)
                lr.start()
                started.append(lr)

            compute_group(lax.rem(my_i - r + N_DEV, N_DEV), rbuf, rs,
                          first=(r == 0))
            if 1 <= r <= 3:
                compute_group(lax.rem(my_i + r, N_DEV), lbuf, rs, first=False)

            for rd in started:
                rd.wait()
            if r <= 2:
                pl.semaphore_signal(rcred, inc=1, device_id=(left,),
                                    device_id_type=_MESH)
            if r <= 1:
                pl.semaphore_signal(lcred, inc=1, device_id=(right,),
                                    device_id_type=_MESH)

    return pl.pallas_call(
        body,
        out_shape=jax.ShapeDtypeStruct((B_LOC, SQ, DM), jnp.float32),
        in_specs=[pl.BlockSpec(memory_space=pltpu.VMEM)] * 4,
        out_specs=pl.BlockSpec(memory_space=pltpu.VMEM),
        scratch_shapes=[
            pltpu.VMEM((2, 2, H_LOC * DH, DM), jnp.bfloat16),
            pltpu.VMEM((2, 2, H_LOC * DH, DM), jnp.bfloat16),
            pltpu.VMEM((H_LOC * DH, B_LOC * SQ), jnp.bfloat16),
            pltpu.VMEM((H_LOC * DH, B_LOC * SQ), jnp.bfloat16),
            pltpu.SemaphoreType.DMA((2,)),
            pltpu.SemaphoreType.DMA((2,)),
            pltpu.SemaphoreType.DMA((2,)),
            pltpu.SemaphoreType.DMA((2,)),
            pltpu.SemaphoreType.REGULAR,
            pltpu.SemaphoreType.REGULAR,
        ],
        compiler_params=pltpu.CompilerParams(
            collective_id=0,
            vmem_limit_bytes=62 * 1024 * 1024,
        ),
    )(x_s, wqo, k_t, v_t)


# device time: 132952 ns/iter; 2.4853x vs baseline; 1.0085x over previous
import jax
import jax.numpy as jnp
from jax import lax
from jax.experimental import pallas as pl
from jax.experimental.pallas import tpu as pltpu

N_DEV = 8
B_LOC = 2
SQ = 512
SKV = 512
H_LOC = 8
DH = 64
DM = 768

_MESH = pl.DeviceIdType.MESH


def kernel(x, Wq, K_ext, V_ext, Wo):
    my = lax.axis_index("i")
    x_s = (x * jnp.float32(0.125)).astype(jnp.bfloat16)
    wq_t = Wq.reshape(DM, H_LOC * DH).T
    wqo = jnp.stack([wq_t, Wo]).astype(jnp.bfloat16)
    k_loc = lax.dynamic_slice_in_dim(K_ext, B_LOC * my, B_LOC, 0)
    v_loc = lax.dynamic_slice_in_dim(V_ext, B_LOC * my, B_LOC, 0)
    k_t = jnp.transpose(k_loc, (2, 0, 3, 1)).astype(jnp.bfloat16)
    v_t = jnp.transpose(v_loc, (2, 0, 3, 1)).astype(jnp.bfloat16)
    v_t = jnp.concatenate(
        [v_t, jnp.ones((64, B_LOC, 1, SKV), jnp.bfloat16)], axis=2)

    def body(x_ref, wqo_ref, k_ref, v_ref, out_ref,
             rbuf, lbuf, qs, cx, rsend, rrecv, lsend, lrecv):
        my_i = lax.axis_index("i")
        left = lax.rem(my_i + N_DEV - 1, N_DEV)
        right = lax.rem(my_i + 1, N_DEV)

        barrier = pltpu.get_barrier_semaphore()
        for nbr in (left, right):
            pl.semaphore_signal(barrier, inc=1, device_id=(nbr,),
                                device_id_type=_MESH)
        pl.semaphore_wait(barrier, 2)

        rbuf[0] = wqo_ref[...]
        lbuf[0] = wqo_ref[...]

        qb = lax.broadcasted_iota(jnp.int32, (SQ, SKV), 0) // 64
        kb = lax.broadcasted_iota(jnp.int32, (SQ, SKV), 1) // 64
        keep = (qb == kb) | (kb == 0) | (lax.rem(qb + kb, 3) == 0)
        bias = jnp.where(keep, jnp.float32(0.0), jnp.float32(-1e9))

        def compute_group(grp, wqv, wov, first):
            x_all = x_ref[...].reshape(B_LOC * SQ, DM)
            qs[...] = lax.dot_general(
                wqv, x_all, (((1,), (1,)), ((), ())),
                preferred_element_type=jnp.float32,
            ).astype(jnp.bfloat16)

            def head(hh, carry):
                g = grp * H_LOC + hh
                for b in range(B_LOC):
                    qt = qs[pl.ds(hh * DH, DH), b * SQ:(b + 1) * SQ]
                    s = lax.dot_general(
                        qt, k_ref[g, b], (((0,), (0,)), ((), ())),
                        preferred_element_type=jnp.float32)
                    e = jnp.exp(s + bias)
                    ca = lax.dot_general(
                        v_ref[g, b], e.astype(jnp.bfloat16),
                        (((1,), (1,)), ((), ())),
                        preferred_element_type=jnp.float32)
                    cx[pl.ds(hh * DH, DH), b * SQ:(b + 1) * SQ] = (
                        ca[:DH] / ca[DH:DH + 1]).astype(jnp.bfloat16)
                return carry

            lax.fori_loop(0, H_LOC, head, 0)
            contrib = lax.dot_general(
                cx[...], wov, (((0,), (0,)), ((), ())),
                preferred_element_type=jnp.float32)
            for b in range(B_LOC):
                blk = contrib[b * SQ:(b + 1) * SQ]
                if first:
                    out_ref[b] = blk
                else:
                    out_ref[b] = out_ref[b] + blk

        for r in range(5):
            started = []
            if r <= 2:
                rr = pltpu.make_async_remote_copy(
                    src_ref=rbuf.at[r], dst_ref=rbuf.at[r + 1],
                    send_sem=rsend.at[r], recv_sem=rrecv.at[r + 1],
                    device_id=(right,), device_id_type=_MESH)
                rr.start()
                started.append(rr)
            elif r == 3:
                rr = pltpu.make_async_remote_copy(
                    src_ref=rbuf.at[3, 0], dst_ref=rbuf.at[4, 0],
                    send_sem=rsend.at[3], recv_sem=rrecv.at[4],
                    device_id=(right,), device_id_type=_MESH)
                rr.start()
                started.append(rr)
            if r <= 2:
                lr = pltpu.make_async_remote_copy(
                    src_ref=lbuf.at[r], dst_ref=lbuf.at[r + 1],
                    send_sem=lsend.at[r], recv_sem=lrecv.at[r + 1],
                    device_id=(left,), device_id_type=_MESH)
                lr.start()
                started.append(lr)
            elif r == 3:
                lr = pltpu.make_async_remote_copy(
                    src_ref=lbuf.at[3, 1], dst_ref=lbuf.at[4, 1],
                    send_sem=lsend.at[3], recv_sem=lrecv.at[4],
                    device_id=(left,), device_id_type=_MESH)
                lr.start()
                started.append(lr)

            if r == 0:
                compute_group(my_i, rbuf[0, 0], rbuf[0, 1], first=True)
            elif r <= 3:
                compute_group(lax.rem(my_i - r + N_DEV, N_DEV),
                              rbuf[r, 0], rbuf[r, 1], first=False)
                compute_group(lax.rem(my_i + r, N_DEV),
                              lbuf[r, 0], lbuf[r, 1], first=False)
            else:
                compute_group(lax.rem(my_i + 4, N_DEV),
                              rbuf[4, 0], lbuf[4, 1], first=False)

            for rd in started:
                rd.wait()

    return pl.pallas_call(
        body,
        out_shape=jax.ShapeDtypeStruct((B_LOC, SQ, DM), jnp.float32),
        in_specs=[pl.BlockSpec(memory_space=pltpu.VMEM)] * 4,
        out_specs=pl.BlockSpec(memory_space=pltpu.VMEM),
        scratch_shapes=[
            pltpu.VMEM((5, 2, H_LOC * DH, DM), jnp.bfloat16),
            pltpu.VMEM((5, 2, H_LOC * DH, DM), jnp.bfloat16),
            pltpu.VMEM((H_LOC * DH, B_LOC * SQ), jnp.bfloat16),
            pltpu.VMEM((H_LOC * DH, B_LOC * SQ), jnp.bfloat16),
            pltpu.SemaphoreType.DMA((4,)),
            pltpu.SemaphoreType.DMA((5,)),
            pltpu.SemaphoreType.DMA((4,)),
            pltpu.SemaphoreType.DMA((5,)),
        ],
        compiler_params=pltpu.CompilerParams(
            collective_id=0,
            vmem_limit_bytes=62 * 1024 * 1024,
        ),
    )(x_s, wqo, k_t, v_t)


# device time: 123135 ns/iter; 2.6835x vs baseline; 1.0797x over previous
import jax
import jax.numpy as jnp
from jax import lax
from jax.experimental import pallas as pl
from jax.experimental.pallas import tpu as pltpu

N_DEV = 8
B_LOC = 2
SQ = 512
SKV = 512
H_LOC = 8
DH = 64
DM = 768

_MESH = pl.DeviceIdType.MESH


def kernel(x, Wq, K_ext, V_ext, Wo):
    my = lax.axis_index("i")
    x_s = (x * jnp.float32(0.125)).astype(jnp.bfloat16)
    wq_t = Wq.reshape(DM, H_LOC * DH).T
    wqo = jnp.stack([wq_t, Wo]).astype(jnp.bfloat16)
    k_loc = lax.dynamic_slice_in_dim(K_ext, B_LOC * my, B_LOC, 0)
    v_loc = lax.dynamic_slice_in_dim(V_ext, B_LOC * my, B_LOC, 0)
    k_t = jnp.transpose(k_loc.astype(jnp.bfloat16), (2, 0, 3, 1))
    v_t = jnp.transpose(v_loc.astype(jnp.bfloat16), (2, 0, 3, 1))
    v_t = jnp.concatenate(
        [v_t, jnp.ones((64, B_LOC, 1, SKV), jnp.bfloat16)], axis=2)

    def body(x_ref, wqo_ref, k_ref, v_ref, out_ref,
             rbuf, lbuf, qs, cx, rsend, rrecv, lsend, lrecv):
        my_i = lax.axis_index("i")
        left = lax.rem(my_i + N_DEV - 1, N_DEV)
        right = lax.rem(my_i + 1, N_DEV)

        barrier = pltpu.get_barrier_semaphore()
        for nbr in (left, right):
            pl.semaphore_signal(barrier, inc=1, device_id=(nbr,),
                                device_id_type=_MESH)
        pl.semaphore_wait(barrier, 2)

        rbuf[0] = wqo_ref[...]
        lbuf[0] = wqo_ref[...]

        qb = lax.broadcasted_iota(jnp.int32, (SQ, SKV), 0) // 64
        kb = lax.broadcasted_iota(jnp.int32, (SQ, SKV), 1) // 64
        keep = (qb == kb) | (kb == 0) | (lax.rem(qb + kb, 3) == 0)
        bias = jnp.where(keep, jnp.float32(0.0), jnp.float32(-1e9))

        def compute_group(grp, wqv, wov, first):
            x_all = x_ref[...].reshape(B_LOC * SQ, DM)
            qs[...] = lax.dot_general(
                wqv, x_all, (((1,), (1,)), ((), ())),
                preferred_element_type=jnp.float32,
            ).astype(jnp.bfloat16)

            for hh in range(H_LOC):
                g = grp * H_LOC + hh
                for b in range(B_LOC):
                    qt = qs[hh * DH:(hh + 1) * DH, b * SQ:(b + 1) * SQ]
                    s = lax.dot_general(
                        qt, k_ref[g, b], (((0,), (0,)), ((), ())),
                        preferred_element_type=jnp.float32)
                    e = jnp.exp(s + bias)
                    ca = lax.dot_general(
                        v_ref[g, b], e.astype(jnp.bfloat16),
                        (((1,), (1,)), ((), ())),
                        preferred_element_type=jnp.float32)
                    cx[hh * DH:(hh + 1) * DH, b * SQ:(b + 1) * SQ] = (
                        ca[:DH] / ca[DH:DH + 1]).astype(jnp.bfloat16)
            contrib = lax.dot_general(
                cx[...], wov, (((0,), (0,)), ((), ())),
                preferred_element_type=jnp.float32)
            for b in range(B_LOC):
                blk = contrib[b * SQ:(b + 1) * SQ]
                if first:
                    out_ref[b] = blk
                else:
                    out_ref[b] = out_ref[b] + blk

        for r in range(5):
            started = []
            if r <= 2:
                rr = pltpu.make_async_remote_copy(
                    src_ref=rbuf.at[r], dst_ref=rbuf.at[r + 1],
                    send_sem=rsend.at[r], recv_sem=rrecv.at[r + 1],
                    device_id=(right,), device_id_type=_MESH)
                rr.start()
                started.append(rr)
            elif r == 3:
                rr = pltpu.make_async_remote_copy(
                    src_ref=rbuf.at[3, 0], dst_ref=rbuf.at[4, 0],
                    send_sem=rsend.at[3], recv_sem=rrecv.at[4],
                    device_id=(right,), device_id_type=_MESH)
                rr.start()
                started.append(rr)
            if r <= 2:
                lr = pltpu.make_async_remote_copy(
                    src_ref=lbuf.at[r], dst_ref=lbuf.at[r + 1],
                    send_sem=lsend.at[r], recv_sem=lrecv.at[r + 1],
                    device_id=(left,), device_id_type=_MESH)
                lr.start()
                started.append(lr)
            elif r == 3:
                lr = pltpu.make_async_remote_copy(
                    src_ref=lbuf.at[3, 1], dst_ref=lbuf.at[4, 1],
                    send_sem=lsend.at[3], recv_sem=lrecv.at[4],
                    device_id=(left,), device_id_type=_MESH)
                lr.start()
                started.append(lr)

            if r == 0:
                compute_group(my_i, rbuf[0, 0], rbuf[0, 1], first=True)
            elif r <= 3:
                compute_group(lax.rem(my_i - r + N_DEV, N_DEV),
                              rbuf[r, 0], rbuf[r, 1], first=False)
                compute_group(lax.rem(my_i + r, N_DEV),
                              lbuf[r, 0], lbuf[r, 1], first=False)
            else:
                compute_group(lax.rem(my_i + 4, N_DEV),
                              rbuf[4, 0], lbuf[4, 1], first=False)

            for rd in started:
                rd.wait()

    return pl.pallas_call(
        body,
        out_shape=jax.ShapeDtypeStruct((B_LOC, SQ, DM), jnp.float32),
        in_specs=[pl.BlockSpec(memory_space=pltpu.VMEM)] * 4,
        out_specs=pl.BlockSpec(memory_space=pltpu.VMEM),
        scratch_shapes=[
            pltpu.VMEM((5, 2, H_LOC * DH, DM), jnp.bfloat16),
            pltpu.VMEM((5, 2, H_LOC * DH, DM), jnp.bfloat16),
            pltpu.VMEM((H_LOC * DH, B_LOC * SQ), jnp.bfloat16),
            pltpu.VMEM((H_LOC * DH, B_LOC * SQ), jnp.bfloat16),
            pltpu.SemaphoreType.DMA((4,)),
            pltpu.SemaphoreType.DMA((5,)),
            pltpu.SemaphoreType.DMA((4,)),
            pltpu.SemaphoreType.DMA((5,)),
        ],
        compiler_params=pltpu.CompilerParams(
            collective_id=0,
            vmem_limit_bytes=62 * 1024 * 1024,
        ),
    )(x_s, wqo, k_t, v_t)


# device time: 121670 ns/iter; 2.7158x vs baseline; 1.0120x over previous
import jax
import jax.numpy as jnp
from jax import lax
from jax.experimental import pallas as pl
from jax.experimental.pallas import tpu as pltpu

N_DEV = 8
B_LOC = 2
SQ = 512
SKV = 512
H_LOC = 8
DH = 64
DM = 768

_MESH = pl.DeviceIdType.MESH


def kernel(x, Wq, K_ext, V_ext, Wo):
    my = lax.axis_index("i")
    x_s = (x * jnp.float32(0.125)).astype(jnp.bfloat16)
    wq_t = Wq.reshape(DM, H_LOC * DH).T
    wqo = jnp.stack([wq_t, Wo]).astype(jnp.bfloat16)
    k_loc = lax.dynamic_slice_in_dim(K_ext, B_LOC * my, B_LOC, 0)
    v_loc = lax.dynamic_slice_in_dim(V_ext, B_LOC * my, B_LOC, 0)
    k_t = jnp.transpose(k_loc.astype(jnp.bfloat16), (2, 0, 3, 1))
    v_t = jnp.transpose(v_loc.astype(jnp.bfloat16), (2, 0, 3, 1))
    v_t = jnp.concatenate(
        [v_t, jnp.ones((64, B_LOC, 1, SKV), jnp.bfloat16)], axis=2)

    def body(x_ref, wqo_ref, k_ref, v_ref, out_ref,
             rbuf, lbuf, qs, cx, rsend, rrecv, lsend, lrecv):
        my_i = lax.axis_index("i")
        left = lax.rem(my_i + N_DEV - 1, N_DEV)
        right = lax.rem(my_i + 1, N_DEV)

        barrier = pltpu.get_barrier_semaphore()
        for nbr in (left, right):
            pl.semaphore_signal(barrier, inc=1, device_id=(nbr,),
                                device_id_type=_MESH)
        pl.semaphore_wait(barrier, 2)

        qb = lax.broadcasted_iota(jnp.int32, (SQ, SKV), 0) // 64
        kb = lax.broadcasted_iota(jnp.int32, (SQ, SKV), 1) // 64
        keep = (qb == kb) | (kb == 0) | (lax.rem(qb + kb, 3) == 0)
        bias = jnp.where(keep, jnp.float32(0.0), jnp.float32(-1e9))

        def compute_group(grp, wqv, wov, first):
            x_all = x_ref[...].reshape(B_LOC * SQ, DM)
            qs[...] = lax.dot_general(
                wqv, x_all, (((1,), (1,)), ((), ())),
                preferred_element_type=jnp.float32,
            ).astype(jnp.bfloat16)

            for hh in range(H_LOC):
                g = grp * H_LOC + hh
                for b in range(B_LOC):
                    qt = qs[hh * DH:(hh + 1) * DH, b * SQ:(b + 1) * SQ]
                    s = lax.dot_general(
                        qt, k_ref[g, b], (((0,), (0,)), ((), ())),
                        preferred_element_type=jnp.float32)
                    e = jnp.exp(s + bias)
                    ca = lax.dot_general(
                        v_ref[g, b], e.astype(jnp.bfloat16),
                        (((1,), (1,)), ((), ())),
                        preferred_element_type=jnp.float32)
                    cx[hh * DH:(hh + 1) * DH, b * SQ:(b + 1) * SQ] = (
                        ca[:DH] / ca[DH:DH + 1]).astype(jnp.bfloat16)
            contrib = lax.dot_general(
                cx[...], wov, (((0,), (0,)), ((), ())),
                preferred_element_type=jnp.float32)
            for b in range(B_LOC):
                blk = contrib[b * SQ:(b + 1) * SQ]
                if first:
                    out_ref[b] = blk
                else:
                    out_ref[b] = out_ref[b] + blk

        for r in range(5):
            started = []
            if r <= 2:
                rr = pltpu.make_async_remote_copy(
                    src_ref=wqo_ref if r == 0 else rbuf.at[r],
                    dst_ref=rbuf.at[r + 1],
                    send_sem=rsend.at[r], recv_sem=rrecv.at[r + 1],
                    device_id=(right,), device_id_type=_MESH)
                rr.start()
                started.append(rr)
            elif r == 3:
                rr = pltpu.make_async_remote_copy(
                    src_ref=rbuf.at[3, 0], dst_ref=rbuf.at[4, 0],
                    send_sem=rsend.at[3], recv_sem=rrecv.at[4],
                    device_id=(right,), device_id_type=_MESH)
                rr.start()
                started.append(rr)
            if r <= 2:
                lr = pltpu.make_async_remote_copy(
                    src_ref=wqo_ref if r == 0 else lbuf.at[r],
                    dst_ref=lbuf.at[r + 1],
                    send_sem=lsend.at[r], recv_sem=lrecv.at[r + 1],
                    device_id=(left,), device_id_type=_MESH)
                lr.start()
                started.append(lr)
            elif r == 3:
                lr = pltpu.make_async_remote_copy(
                    src_ref=lbuf.at[3, 1], dst_ref=lbuf.at[4, 1],
                    send_sem=lsend.at[3], recv_sem=lrecv.at[4],
                    device_id=(left,), device_id_type=_MESH)
                lr.start()
                started.append(lr)

            if r == 0:
                compute_group(my_i, wqo_ref[0], wqo_ref[1], first=True)
            elif r <= 3:
                compute_group(lax.rem(my_i - r + N_DEV, N_DEV),
                              rbuf[r, 0], rbuf[r, 1], first=False)
                compute_group(lax.rem(my_i + r, N_DEV),
                              lbuf[r, 0], lbuf[r, 1], first=False)
            else:
                compute_group(lax.rem(my_i + 4, N_DEV),
                              rbuf[4, 0], lbuf[4, 1], first=False)

            for rd in started:
                rd.wait()

    return pl.pallas_call(
        body,
        out_shape=jax.ShapeDtypeStruct((B_LOC, SQ, DM), jnp.float32),
        in_specs=[pl.BlockSpec(memory_space=pltpu.VMEM)] * 4,
        out_specs=pl.BlockSpec(memory_space=pltpu.VMEM),
        scratch_shapes=[
            pltpu.VMEM((5, 2, H_LOC * DH, DM), jnp.bfloat16),
            pltpu.VMEM((5, 2, H_LOC * DH, DM), jnp.bfloat16),
            pltpu.VMEM((H_LOC * DH, B_LOC * SQ), jnp.bfloat16),
            pltpu.VMEM((H_LOC * DH, B_LOC * SQ), jnp.bfloat16),
            pltpu.SemaphoreType.DMA((4,)),
            pltpu.SemaphoreType.DMA((5,)),
            pltpu.SemaphoreType.DMA((4,)),
            pltpu.SemaphoreType.DMA((5,)),
        ],
        compiler_params=pltpu.CompilerParams(
            collective_id=0,
            vmem_limit_bytes=62 * 1024 * 1024,
        ),
    )(x_s, wqo, k_t, v_t)
